# Initial kernel scaffold; baseline (speedup 1.0000x reference)
import jax
import jax.numpy as jnp
from jax.experimental import pallas as pl


def kernel(x, edge_index, W, b):
    raise NotImplementedError("write your pallas kernel here")


# speedup vs baseline: 10.3724x; 10.3724x over previous
import functools

import jax
import jax.numpy as jnp
from jax import lax
from jax.experimental import pallas as pl
from jax.experimental.pallas import tpu as pltpu
from jax.experimental.pallas import tpu_sc as plsc

N = 10000
E = 160000
D_IN = 256
D_OUT = 256
HALF = 128
NC = 2
NS = 16
LANES = 16
N_PAD = 10240
NPS = N_PAD // NS
CH = 128
NCHUNKS = E // CH
ROW_BLK = 512

_mesh = plsc.VectorSubcoreMesh(
    core_axis_name="c", subcore_axis_name="s", num_cores=NC, num_subcores=NS)


@functools.partial(
    pl.kernel,
    out_type=jax.ShapeDtypeStruct((NC, N_PAD), jnp.float32),
    mesh=_mesh,
    scratch_types=[
        pltpu.VMEM((CH,), jnp.int32),
        pltpu.VMEM((CH,), jnp.float32),
        pltpu.VMEM((NPS,), jnp.float32),
        pltpu.VMEM_SHARED((N_PAD,), jnp.float32),
    ],
)
def _deg(ei, dega, colb, onesb, stage, hist):
    c = lax.axis_index("c")
    s = lax.axis_index("s")
    one_v = jnp.full((LANES,), 1.0, jnp.float32)
    for v in range(CH // LANES):
        onesb[pl.ds(v * LANES, LANES)] = one_v
    zero_v = jnp.zeros((LANES,), jnp.float32)

    def _z(i, carry):
        stage[pl.ds(i * LANES, LANES)] = zero_v
        return carry

    lax.fori_loop(0, NPS // LANES, _z, 0)
    pltpu.sync_copy(stage, hist.at[pl.ds(s * NPS, NPS)])
    plsc.subcore_barrier()

    half_chunks = NCHUNKS // NC
    nch = 39 + jnp.where(s < 1, 1, 0)

    def _body(j, carry):
        i = c * half_chunks + s + j * NS
        pltpu.sync_copy(ei.at[1, pl.ds(i * CH, CH)], colb)
        pltpu.sync_copy(onesb, hist.at[colb], add=True)
        return carry

    lax.fori_loop(0, nch, _body, 0)
    plsc.subcore_barrier()
    pltpu.sync_copy(hist.at[pl.ds(s * NPS, NPS)], stage)
    pltpu.sync_copy(stage, dega.at[c, pl.ds(s * NPS, NPS)])


def _mm_body(x_ref, w_ref, degt_ref, y_ref, dis_ref):
    deg = degt_ref[:, 0:1] + degt_ref[:, 1:2] + 1.0
    dis = lax.rsqrt(deg)
    xw = jnp.dot(x_ref[...], w_ref[...],
                 preferred_element_type=jnp.float32,
                 precision=lax.Precision.HIGHEST)
    y_ref[...] = xw * dis
    dis_ref[...] = dis


_mm = pl.pallas_call(
    _mm_body,
    grid=(NC, N_PAD // ROW_BLK),
    in_specs=[
        pl.BlockSpec((ROW_BLK, D_IN), lambda c, i: (i, 0)),
        pl.BlockSpec((D_IN, HALF), lambda c, i: (0, c)),
        pl.BlockSpec((ROW_BLK, 2), lambda c, i: (i, 0)),
    ],
    out_specs=[
        pl.BlockSpec((ROW_BLK, HALF),
                     lambda c, i: (c * (N_PAD // ROW_BLK) + i, 0)),
        pl.BlockSpec((ROW_BLK, 1), lambda c, i: (i, 0)),
    ],
    out_shape=[
        jax.ShapeDtypeStruct((NC * N_PAD, HALF), jnp.float32),
        jax.ShapeDtypeStruct((N_PAD, 1), jnp.float32),
    ],
)


@functools.partial(
    pl.kernel,
    out_type=jax.ShapeDtypeStruct((NC, N_PAD, HALF), jnp.float32),
    mesh=_mesh,
    scratch_types=[
        pltpu.VMEM((CH,), jnp.int32),
        pltpu.VMEM((CH,), jnp.int32),
        pltpu.VMEM((CH, HALF), jnp.float32),
        pltpu.VMEM((CH, HALF), jnp.float32),
        pltpu.VMEM((NPS,), jnp.float32),
        pltpu.VMEM((HALF,), jnp.float32),
        pltpu.VMEM_SHARED((N_PAD, HALF), jnp.float32),
        pltpu.SemaphoreType.DMA,
    ],
)
def _msg(ei, y, dis, b, outp, rowb, colb, gbuf, obuf, disv, bb, acc, sem):
    c = lax.axis_index("c")
    s = lax.axis_index("s")
    n0 = s * NPS
    for k in range(NPS // CH):
        pltpu.sync_copy(y.at[pl.ds(c * N_PAD + n0 + k * CH, CH)], gbuf)
        pltpu.sync_copy(gbuf, acc.at[pl.ds(n0 + k * CH, CH)])
    plsc.subcore_barrier()

    nch = 78 + jnp.where(s < 2, 1, 0)
    off = c * N_PAD

    def _body(j, carry):
        i = s + j * NS
        pltpu.sync_copy(ei.at[0, pl.ds(i * CH, CH)], rowb)
        pltpu.sync_copy(ei.at[1, pl.ds(i * CH, CH)], colb)
        for v in range(CH // LANES):
            sl = pl.ds(v * LANES, LANES)
            rowb[sl] = rowb[sl] + off
        pltpu.async_copy(y.at[rowb], gbuf, sem).wait()
        pltpu.sync_copy(gbuf, acc.at[colb], add=True)
        return carry

    lax.fori_loop(0, nch, _body, 0)
    plsc.subcore_barrier()

    pltpu.sync_copy(dis.at[pl.ds(n0, NPS)], disv)
    pltpu.sync_copy(b.at[pl.ds(c * HALF, HALF)], bb)
    for k in range(NPS // CH):
        pltpu.sync_copy(acc.at[pl.ds(n0 + k * CH, CH)], gbuf)

        def _fin(g, carry, k=k):
            dvec = disv[pl.ds(k * CH + g * LANES, LANES)]
            for i in range(LANES):
                nn = g * LANES + i
                dval = dvec[i]
                for v in range(HALF // LANES):
                    sl = pl.ds(v * LANES, LANES)
                    obuf[nn, sl] = jnp.maximum(
                        gbuf[nn, sl] * dval + bb[sl], 0.0)
            return carry

        lax.fori_loop(0, CH // LANES, _fin, 0)
        pltpu.sync_copy(obuf, outp.at[c, pl.ds(n0 + k * CH, CH)])


def kernel(x, edge_index, W, b):
    x_pad = jnp.pad(x, ((0, N_PAD - N), (0, 0)))
    dega = _deg(edge_index)
    y, dis = _mm(x_pad, W, dega.T)
    outp = _msg(edge_index, y, dis.reshape(N_PAD), b)
    return outp.transpose(1, 0, 2).reshape(N_PAD, D_OUT)[:N]


# speedup vs baseline: 16.9377x; 1.6330x over previous
import functools

import jax
import jax.numpy as jnp
from jax import lax
from jax.experimental import pallas as pl
from jax.experimental.pallas import tpu as pltpu
from jax.experimental.pallas import tpu_sc as plsc

N = 10000
E = 160000
D_IN = 256
D_OUT = 256
HALF = 128
NC = 2
NS = 16
LANES = 16
N_PAD = 10240
NPS = N_PAD // NS
CH = 128
NCHUNKS = E // CH
MAXCH = 80
DEGCH = 40
ROW_BLK = 512

_mesh = plsc.VectorSubcoreMesh(
    core_axis_name="c", subcore_axis_name="s", num_cores=NC, num_subcores=NS)


@functools.partial(
    pl.kernel,
    out_type=jax.ShapeDtypeStruct((NC, N_PAD), jnp.float32),
    mesh=_mesh,
    scratch_types=[
        pltpu.VMEM((DEGCH, CH), jnp.int32),
        pltpu.VMEM((CH,), jnp.float32),
        pltpu.VMEM((NPS,), jnp.float32),
        pltpu.VMEM_SHARED((N_PAD,), jnp.float32),
        pltpu.SemaphoreType.DMA,
    ],
)
def _deg(ei3, dega, cols2d, onesb, stage, hist, dsem):
    c = lax.axis_index("c")
    s = lax.axis_index("s")
    one_v = jnp.full((LANES,), 1.0, jnp.float32)
    for v in range(CH // LANES):
        onesb[pl.ds(v * LANES, LANES)] = one_v
    zero_v = jnp.zeros((LANES,), jnp.float32)

    def _z(i, carry):
        stage[pl.ds(i * LANES, LANES)] = zero_v
        return carry

    lax.fori_loop(0, NPS // LANES, _z, 0)
    pltpu.sync_copy(stage, hist.at[pl.ds(s * NPS, NPS)])
    plsc.subcore_barrier()

    w = c * NS + s
    five = w < 28
    gstart = jnp.where(five, 5 * w, 140 + 4 * (w - 28))
    start = gstart * 8
    nch = jnp.where(w == 31, 34, jnp.where(five, 40, 32))

    @pl.when(five)
    def _():
        pltpu.sync_copy(ei3.at[1, pl.ds(start, 40)], cols2d.at[pl.ds(0, 40)])

    @pl.when(jnp.logical_not(five))
    def _():
        pltpu.sync_copy(ei3.at[1, pl.ds(start, 32)], cols2d.at[pl.ds(0, 32)])

    @pl.when(w == 31)
    def _():
        pltpu.sync_copy(ei3.at[1, pl.ds(1248, 2)], cols2d.at[pl.ds(32, 2)])

    for j in range(DEGCH):
        @pl.when(j < nch)
        def _(j=j):
            pltpu.make_async_copy(
                onesb, hist.at[cols2d.at[j]], dsem).start(add=True)
    for j in range(DEGCH):
        @pl.when(j < nch)
        def _(j=j):
            pltpu.make_async_copy(onesb, hist.at[cols2d.at[0]], dsem).wait()

    plsc.subcore_barrier()
    pltpu.sync_copy(hist.at[pl.ds(s * NPS, NPS)], stage)
    pltpu.sync_copy(stage, dega.at[c, pl.ds(s * NPS, NPS)])


def _mm_body(x_ref, w_ref, degt_ref, y_ref, dis_ref):
    deg = degt_ref[:, 0:1] + degt_ref[:, 1:2] + 1.0
    dis = lax.rsqrt(deg)
    xw = jnp.dot(x_ref[...], w_ref[...],
                 preferred_element_type=jnp.float32,
                 precision=lax.Precision.HIGHEST)
    y_ref[...] = xw * dis
    dis_ref[...] = dis


_mm = pl.pallas_call(
    _mm_body,
    grid=(NC, N_PAD // ROW_BLK),
    in_specs=[
        pl.BlockSpec((ROW_BLK, D_IN), lambda c, i: (i, 0)),
        pl.BlockSpec((D_IN, HALF), lambda c, i: (0, c)),
        pl.BlockSpec((ROW_BLK, 2), lambda c, i: (i, 0)),
    ],
    out_specs=[
        pl.BlockSpec((ROW_BLK, HALF),
                     lambda c, i: (c * (N_PAD // ROW_BLK) + i, 0)),
        pl.BlockSpec((ROW_BLK, 1), lambda c, i: (i, 0)),
    ],
    out_shape=[
        jax.ShapeDtypeStruct((NC * N_PAD, HALF), jnp.float32),
        jax.ShapeDtypeStruct((N_PAD, 1), jnp.float32),
    ],
)


MCH = 64
MPC = 40
MPHASES = 4
MSG_CHUNKS = 2504
NB = NPS // MCH


@functools.partial(
    pl.kernel,
    out_type=jax.ShapeDtypeStruct((NC, N_PAD, HALF), jnp.float32),
    mesh=_mesh,
    scratch_types=[
        pltpu.VMEM((MPC, MCH), jnp.int32),
        pltpu.VMEM((MPC, MCH), jnp.int32),
        pltpu.VMEM((4, MCH, HALF), jnp.float32),
        pltpu.VMEM((NPS,), jnp.float32),
        pltpu.VMEM((HALF,), jnp.float32),
        pltpu.VMEM_SHARED((N_PAD, HALF), jnp.float32),
        pltpu.SemaphoreType.DMA((4,)),
        pltpu.SemaphoreType.DMA((4,)),
    ],
)
def _msg(ei3, y, dis, b, outp, rows2d, cols2d, gbuf, disv, bb, acc,
         gsem, ssem):
    c = lax.axis_index("c")
    s = lax.axis_index("s")
    n0 = s * NPS

    def _yload(k, d):
        return pltpu.make_async_copy(
            y.at[pl.ds(c * N_PAD + n0 + k * MCH, MCH)], gbuf.at[d],
            gsem.at[d])

    _yload(0, 0).start()
    _yload(1, 1).start()
    for k in range(NB):
        d = k % 2
        _yload(k, d).wait()
        pltpu.sync_copy(gbuf.at[d], acc.at[pl.ds(n0 + k * MCH, MCH)])
        if k + 2 < NB:
            _yload(k + 2, d).start()

    nch = jnp.where(s < 8, 160, jnp.where(s < 15, 152, 156))
    start = jnp.where(s < 8, 160 * s, 1280 + 152 * (s - 8))
    off = c * N_PAD
    plsc.subcore_barrier()

    def _gather(j, d):
        return pltpu.make_async_copy(
            y.at[rows2d.at[j]], gbuf.at[d], gsem.at[d])

    def _scatter(j, d):
        return pltpu.make_async_copy(
            gbuf.at[d], acc.at[cols2d.at[j]], ssem.at[d])

    def _phase(p, carry):
        pb = start + p * MPC
        q0 = p * MPC
        pltpu.sync_copy(ei3.at[0, pl.ds(pb, MPC)], rows2d)
        pltpu.sync_copy(ei3.at[1, pl.ds(pb, MPC)], cols2d)

        def _adj(r, cry):
            for v in range(MCH // LANES):
                sl = pl.ds(v * LANES, LANES)
                rows2d[r, sl] = rows2d[r, sl] + off
            return cry

        lax.fori_loop(0, MPC, _adj, 0)

        for j in range(3):
            @pl.when(q0 + j < nch)
            def _(j=j):
                _gather(j, j % 4).start()

        for j in range(MPC):
            @pl.when(q0 + j < nch)
            def _(j=j):
                d = j % 4
                _gather(j, d).wait()
                _scatter(j, d).start(add=True)
                if j + 3 < MPC:
                    @pl.when(q0 + j + 3 < nch)
                    def _():
                        if j >= 1:
                            _scatter(0, (j - 1) % 4).wait()
                        _gather(j + 3, (j + 3) % 4).start()

        for dd in range(4):
            _scatter(0, dd).wait()
        return carry

    lax.fori_loop(0, MPHASES, _phase, 0)
    plsc.subcore_barrier()

    pltpu.sync_copy(dis.at[pl.ds(n0, NPS)], disv)
    pltpu.sync_copy(b.at[pl.ds(c * HALF, HALF)], bb)

    def _aread(k, d):
        return pltpu.make_async_copy(
            acc.at[pl.ds(n0 + k * MCH, MCH)], gbuf.at[d], gsem.at[d])

    def _owrite(k, d):
        return pltpu.make_async_copy(
            gbuf.at[2 + d], outp.at[c, pl.ds(n0 + k * MCH, MCH)], ssem.at[d])

    _aread(0, 0).start()

    def _finpair(k2, carry):
        for d in range(2):
            k = k2 * 2 + d
            _aread(k, d).wait()

            @pl.when(k + 1 < NB)
            def _(d=d, k=k):
                _aread(k + 1, 1 - d).start()

            @pl.when(k >= 2)
            def _(d=d):
                _owrite(0, d).wait()

            def _fin(g, cry, d=d, k=k):
                dvec = disv[pl.ds(k * MCH + g * LANES, LANES)]
                for i in range(LANES):
                    nn = g * LANES + i
                    dval = dvec[i]
                    for v in range(HALF // LANES):
                        sl = pl.ds(v * LANES, LANES)
                        gbuf[2 + d, nn, sl] = jnp.maximum(
                            gbuf[d, nn, sl] * dval + bb[sl], 0.0)
                return cry

            lax.fori_loop(0, MCH // LANES, _fin, 0)
            _owrite(k, d).start()
        return carry

    lax.fori_loop(0, NB // 2, _finpair, 0)
    _owrite(0, 0).wait()
    _owrite(0, 1).wait()


def kernel(x, edge_index, W, b):
    x_pad = jnp.pad(x, ((0, N_PAD - N), (0, 0)))
    ei3d = edge_index.reshape(2, NCHUNKS, CH)
    ei3m = jnp.pad(edge_index,
                   ((0, 0), (0, MSG_CHUNKS * MCH - E))).reshape(
                       2, MSG_CHUNKS, MCH)
    dega = _deg(ei3d)
    y, dis = _mm(x_pad, W, dega.T)
    outp = _msg(ei3m, y, dis.reshape(N_PAD), b)
    return outp.transpose(1, 0, 2).reshape(N_PAD, D_OUT)[:N]


# speedup vs baseline: 19.1515x; 1.1307x over previous
import functools

import jax
import jax.numpy as jnp
from jax import lax
from jax.experimental import pallas as pl
from jax.experimental.pallas import tpu as pltpu
from jax.experimental.pallas import tpu_sc as plsc

N = 10000
E = 160000
D_IN = 256
D_OUT = 256
HALF = 128
NC = 2
NS = 16
LANES = 16
N_PAD = 10240
NPS = N_PAD // NS
MCH = 64
MSG_CHUNKS = 2504
DEGCH = 80
MPC = 40
MPHASES = 4
NB = 10
ROW_BLK = 400

_mesh = plsc.VectorSubcoreMesh(
    core_axis_name="c", subcore_axis_name="s", num_cores=NC, num_subcores=NS)


@functools.partial(
    pl.kernel,
    out_type=jax.ShapeDtypeStruct((NC, N_PAD), jnp.float32),
    mesh=_mesh,
    scratch_types=[
        pltpu.VMEM((DEGCH, MCH), jnp.int32),
        pltpu.VMEM((MCH,), jnp.float32),
        pltpu.VMEM((NPS,), jnp.float32),
        pltpu.VMEM_SHARED((N_PAD,), jnp.float32),
        pltpu.SemaphoreType.DMA,
    ],
)
def _deg(ei3, dega, cols2d, onesb, stage, hist, dsem):
    c = lax.axis_index("c")
    s = lax.axis_index("s")
    one_v = jnp.full((LANES,), 1.0, jnp.float32)
    for v in range(MCH // LANES):
        onesb[pl.ds(v * LANES, LANES)] = one_v
    zero_v = jnp.zeros((LANES,), jnp.float32)

    def _z(i, carry):
        stage[pl.ds(i * LANES, LANES)] = zero_v
        return carry

    lax.fori_loop(0, NPS // LANES, _z, 0)
    pltpu.sync_copy(stage, hist.at[pl.ds(s * NPS, NPS)])
    plsc.subcore_barrier()

    w = c * NS + s
    ten = w < 24
    start = jnp.where(ten, 80 * w, 1920 + 72 * (w - 24))
    nch = jnp.where(w == 31, 76, jnp.where(ten, 80, 72))

    @pl.when(ten)
    def _():
        pltpu.sync_copy(ei3.at[1, pl.ds(start, 80)], cols2d.at[pl.ds(0, 80)])

    @pl.when(jnp.logical_not(ten))
    def _():
        pltpu.sync_copy(ei3.at[1, pl.ds(start, 72)], cols2d.at[pl.ds(0, 72)])

    @pl.when(w == 31)
    def _():
        pltpu.sync_copy(ei3.at[1, pl.ds(2496, 4)], cols2d.at[pl.ds(72, 4)])

    for j in range(DEGCH):
        @pl.when(j < nch)
        def _(j=j):
            pltpu.make_async_copy(
                onesb, hist.at[cols2d.at[j]], dsem).start(add=True)
    for j in range(DEGCH):
        @pl.when(j < nch)
        def _(j=j):
            pltpu.make_async_copy(onesb, hist.at[cols2d.at[0]], dsem).wait()

    plsc.subcore_barrier()
    pltpu.sync_copy(hist.at[pl.ds(s * NPS, NPS)], stage)
    pltpu.sync_copy(stage, dega.at[c, pl.ds(s * NPS, NPS)])


def _mm_body(x_ref, w_ref, degt_ref, y_ref, dis_ref):
    deg = degt_ref[:, 0:1] + degt_ref[:, 1:2] + 1.0
    dis = lax.rsqrt(deg)
    xw = jnp.dot(x_ref[...], w_ref[...],
                 preferred_element_type=jnp.float32,
                 precision=lax.Precision.HIGHEST)
    y_ref[0] = xw * dis
    dis_ref[...] = dis


_mm = pl.pallas_call(
    _mm_body,
    grid=(NC, N // ROW_BLK),
    in_specs=[
        pl.BlockSpec((ROW_BLK, D_IN), lambda c, i: (i, 0)),
        pl.BlockSpec((D_IN, HALF), lambda c, i: (0, c)),
        pl.BlockSpec((ROW_BLK, 2), lambda c, i: (i, 0)),
    ],
    out_specs=[
        pl.BlockSpec((1, ROW_BLK, HALF), lambda c, i: (c, i, 0)),
        pl.BlockSpec((ROW_BLK, 1), lambda c, i: (i, 0)),
    ],
    out_shape=[
        jax.ShapeDtypeStruct((NC, N_PAD, HALF), jnp.float32),
        jax.ShapeDtypeStruct((N_PAD, 1), jnp.float32),
    ],
)


@functools.partial(
    pl.kernel,
    out_type=jax.ShapeDtypeStruct((N, D_OUT), jnp.float32),
    mesh=_mesh,
    scratch_types=[
        pltpu.VMEM((MPC, MCH), jnp.int32),
        pltpu.VMEM((MPC, MCH), jnp.int32),
        pltpu.VMEM((4, MCH, HALF), jnp.float32),
        pltpu.VMEM((NPS,), jnp.float32),
        pltpu.VMEM((HALF,), jnp.float32),
        pltpu.VMEM_SHARED((N_PAD, HALF), jnp.float32),
        pltpu.SemaphoreType.DMA((4,)),
        pltpu.SemaphoreType.DMA((4,)),
    ],
)
def _msg(ei3, y, dis, b, outp, rows2d, cols2d, gbuf, disv, bb, acc,
         gsem, ssem):
    c = lax.axis_index("c")
    s = lax.axis_index("s")
    n0 = s * NPS

    def _yload(k, d):
        return pltpu.make_async_copy(
            y.at[pl.ds(c * N_PAD + n0 + k * MCH, MCH)], gbuf.at[d],
            gsem.at[d])

    _yload(0, 0).start()
    _yload(1, 1).start()
    for k in range(NB):
        d = k % 2
        _yload(k, d).wait()
        pltpu.sync_copy(gbuf.at[d], acc.at[pl.ds(n0 + k * MCH, MCH)])
        if k + 2 < NB:
            _yload(k + 2, d).start()

    nch = jnp.where(s < 8, 160, jnp.where(s < 15, 152, 156))
    start = jnp.where(s < 8, 160 * s, 1280 + 152 * (s - 8))
    off = c * N_PAD
    plsc.subcore_barrier()

    def _gather(j, d):
        return pltpu.make_async_copy(
            y.at[rows2d.at[j]], gbuf.at[d], gsem.at[d])

    def _scatter(j, d):
        return pltpu.make_async_copy(
            gbuf.at[d], acc.at[cols2d.at[j]], ssem.at[d])

    def _phase(p, carry):
        pb = start + p * MPC
        q0 = p * MPC
        pltpu.sync_copy(ei3.at[0, pl.ds(pb, MPC)], rows2d)
        pltpu.sync_copy(ei3.at[1, pl.ds(pb, MPC)], cols2d)

        def _adj(r, cry):
            for v in range(MCH // LANES):
                sl = pl.ds(v * LANES, LANES)
                rows2d[r, sl] = rows2d[r, sl] + off
            return cry

        lax.fori_loop(0, MPC, _adj, 0)

        for j in range(3):
            @pl.when(q0 + j < nch)
            def _(j=j):
                _gather(j, j % 4).start()

        for j in range(MPC):
            @pl.when(q0 + j < nch)
            def _(j=j):
                d = j % 4
                _gather(j, d).wait()
                _scatter(j, d).start(add=True)
                if j + 3 < MPC:
                    @pl.when(q0 + j + 3 < nch)
                    def _():
                        if j >= 1:
                            _scatter(0, (j - 1) % 4).wait()
                        _gather(j + 3, (j + 3) % 4).start()

        for dd in range(4):
            _scatter(0, dd).wait()
        return carry

    lax.fori_loop(0, MPHASES, _phase, 0)
    plsc.subcore_barrier()

    pltpu.sync_copy(dis.at[pl.ds(n0, NPS)], disv)
    pltpu.sync_copy(b.at[pl.ds(c * HALF, HALF)], bb)

    def _aread(k, d):
        return pltpu.make_async_copy(
            acc.at[pl.ds(n0 + k * MCH, MCH)], gbuf.at[d], gsem.at[d])

    def _owrite_full(k, d):
        return pltpu.make_async_copy(
            gbuf.at[2 + d],
            outp.at[pl.ds(n0 + k * MCH, MCH), pl.ds(c * HALF, HALF)],
            ssem.at[d])

    def _owrite_part(k, d):
        return pltpu.make_async_copy(
            gbuf.at[2 + d, pl.ds(0, N % MCH)],
            outp.at[pl.ds(n0 + k * MCH, N % MCH), pl.ds(c * HALF, HALF)],
            ssem.at[d])

    def _ostart(k, d):
        ws = n0 + k * MCH

        @pl.when(ws + MCH <= N)
        def _():
            _owrite_full(k, d).start()

        @pl.when(jnp.logical_and(ws < N, ws + MCH > N))
        def _():
            _owrite_part(k, d).start()

    def _owait(k, d):
        ws = n0 + k * MCH

        @pl.when(ws + MCH <= N)
        def _():
            _owrite_full(k, d).wait()

        @pl.when(jnp.logical_and(ws < N, ws + MCH > N))
        def _():
            _owrite_part(k, d).wait()

    _aread(0, 0).start()

    def _finpair(k2, carry):
        for d in range(2):
            k = k2 * 2 + d
            _aread(k, d).wait()

            @pl.when(k + 1 < NB)
            def _(d=d, k=k):
                _aread(k + 1, 1 - d).start()

            @pl.when(k >= 2)
            def _(d=d, k=k):
                _owait(k - 2, d)

            def _fin(g, cry, d=d, k=k):
                dvec = disv[pl.ds(k * MCH + g * LANES, LANES)]
                for i in range(LANES):
                    nn = g * LANES + i
                    dval = dvec[i]
                    for v in range(HALF // LANES):
                        sl = pl.ds(v * LANES, LANES)
                        gbuf[2 + d, nn, sl] = jnp.maximum(
                            gbuf[d, nn, sl] * dval + bb[sl], 0.0)
                return cry

            lax.fori_loop(0, MCH // LANES, _fin, 0)
            _ostart(k, d)
        return carry

    lax.fori_loop(0, NB // 2, _finpair, 0)
    _owait(NB - 2, 0)
    _owait(NB - 1, 1)


def kernel(x, edge_index, W, b):
    ei3 = jnp.pad(edge_index,
                  ((0, 0), (0, MSG_CHUNKS * MCH - E))).reshape(
                      2, MSG_CHUNKS, MCH)
    dega = _deg(ei3)
    y3, dis = _mm(x, W, dega.T)
    y = y3.reshape(NC * N_PAD, HALF)
    return _msg(ei3, y, dis.reshape(N_PAD), b)


# speedup vs baseline: 19.2194x; 1.0035x over previous
import functools

import jax
import jax.numpy as jnp
import numpy as np
from jax import lax
from jax.experimental import pallas as pl
from jax.experimental.pallas import tpu as pltpu
from jax.experimental.pallas import tpu_sc as plsc

N = 10000
E = 160000
D_IN = 256
D_OUT = 256
HALF = 128
NC = 2
NS = 16
LANES = 16
N_PAD = 10240
NPS = N_PAD // NS
MCH = 64
MSG_CHUNKS = 2504
DEGCH = 80
MPC = 40
MPHASES = 4
NB = 10
ROW_BLK = 400

_mesh = plsc.VectorSubcoreMesh(
    core_axis_name="c", subcore_axis_name="s", num_cores=NC, num_subcores=NS)


@functools.partial(
    pl.kernel,
    out_type=jax.ShapeDtypeStruct((NC, N_PAD), jnp.float32),
    mesh=_mesh,
    scratch_types=[
        pltpu.VMEM((DEGCH, MCH), jnp.int32),
        pltpu.VMEM((MCH,), jnp.float32),
        pltpu.VMEM((NPS,), jnp.float32),
        pltpu.VMEM_SHARED((N_PAD,), jnp.float32),
        pltpu.SemaphoreType.DMA,
    ],
)
def _deg(ei3, dega, cols2d, onesb, stage, hist, dsem):
    c = lax.axis_index("c")
    s = lax.axis_index("s")
    one_v = jnp.full((LANES,), 1.0, jnp.float32)
    for v in range(MCH // LANES):
        onesb[pl.ds(v * LANES, LANES)] = one_v
    zero_v = jnp.zeros((LANES,), jnp.float32)

    def _z(i, carry):
        stage[pl.ds(i * LANES, LANES)] = zero_v
        return carry

    lax.fori_loop(0, NPS // LANES, _z, 0)
    pltpu.sync_copy(stage, hist.at[pl.ds(s * NPS, NPS)])
    plsc.subcore_barrier()

    w = c * NS + s
    ten = w < 24
    start = jnp.where(ten, 80 * w, 1920 + 72 * (w - 24))
    nch = jnp.where(w == 31, 76, jnp.where(ten, 80, 72))

    @pl.when(ten)
    def _():
        pltpu.sync_copy(ei3.at[1, pl.ds(start, 80)], cols2d.at[pl.ds(0, 80)])

    @pl.when(jnp.logical_not(ten))
    def _():
        pltpu.sync_copy(ei3.at[1, pl.ds(start, 72)], cols2d.at[pl.ds(0, 72)])

    @pl.when(w == 31)
    def _():
        pltpu.sync_copy(ei3.at[1, pl.ds(2496, 4)], cols2d.at[pl.ds(72, 4)])

    for j in range(DEGCH):
        @pl.when(j < nch)
        def _(j=j):
            pltpu.make_async_copy(
                onesb, hist.at[cols2d.at[j]], dsem).start(add=True)
    for j in range(DEGCH):
        @pl.when(j < nch)
        def _(j=j):
            pltpu.make_async_copy(onesb, hist.at[cols2d.at[0]], dsem).wait()

    plsc.subcore_barrier()
    pltpu.sync_copy(hist.at[pl.ds(s * NPS, NPS)], stage)
    pltpu.sync_copy(stage, dega.at[c, pl.ds(s * NPS, NPS)])


def _mm_body(x_ref, w_ref, degt_ref, y_ref, dis_ref):
    deg = degt_ref[:, 0:1] + degt_ref[:, 1:2] + 1.0
    dis = lax.rsqrt(deg)
    xw = jnp.dot(x_ref[...], w_ref[...],
                 preferred_element_type=jnp.float32,
                 precision=lax.Precision.HIGHEST)
    y_ref[0] = xw * dis
    dis_ref[...] = dis


_mm = pl.pallas_call(
    _mm_body,
    grid=(NC, N // ROW_BLK),
    in_specs=[
        pl.BlockSpec((ROW_BLK, D_IN), lambda c, i: (i, 0)),
        pl.BlockSpec((D_IN, HALF), lambda c, i: (0, c)),
        pl.BlockSpec((ROW_BLK, 2), lambda c, i: (i, 0)),
    ],
    out_specs=[
        pl.BlockSpec((1, ROW_BLK, HALF), lambda c, i: (c, i, 0)),
        pl.BlockSpec((ROW_BLK, 1), lambda c, i: (i, 0)),
    ],
    out_shape=[
        jax.ShapeDtypeStruct((NC, N_PAD, HALF), jnp.float32),
        jax.ShapeDtypeStruct((N_PAD, 1), jnp.float32),
    ],
)


@functools.partial(
    pl.kernel,
    out_type=jax.ShapeDtypeStruct((N, D_OUT), jnp.float32),
    mesh=_mesh,
    scratch_types=[
        pltpu.VMEM((MPC, MCH), jnp.int32),
        pltpu.VMEM((MPC, MCH), jnp.int32),
        pltpu.VMEM((4, MCH, HALF), jnp.float32),
        pltpu.VMEM((NPS,), jnp.float32),
        pltpu.VMEM((HALF,), jnp.float32),
        pltpu.VMEM_SHARED((N_PAD, HALF), jnp.float32),
        pltpu.SemaphoreType.DMA((4,)),
        pltpu.SemaphoreType.DMA((4,)),
        pltpu.SemaphoreType.DMA((4,)),
    ],
)
def _msg(ei3, y, dis, b, outp, rows2d, cols2d, gbuf, disv, bb, acc,
         gsem, gsem2, ssem):
    c = lax.axis_index("c")
    s = lax.axis_index("s")
    n0 = s * NPS

    def _yload(k, d):
        return pltpu.make_async_copy(
            y.at[pl.ds(c * N_PAD + n0 + k * MCH, MCH)], gbuf.at[d],
            gsem.at[d])

    _yload(0, 0).start()
    _yload(1, 1).start()
    for k in range(NB):
        d = k % 2
        _yload(k, d).wait()
        pltpu.sync_copy(gbuf.at[d], acc.at[pl.ds(n0 + k * MCH, MCH)])
        if k + 2 < NB:
            _yload(k + 2, d).start()

    nch = jnp.where(s < 8, 160, jnp.where(s < 15, 152, 156))
    start = jnp.where(s < 8, 160 * s, 1280 + 152 * (s - 8))
    off = c * N_PAD
    plsc.subcore_barrier()

    def _gather_a(j, d):
        return pltpu.make_async_copy(
            y.at[rows2d.at[j, pl.ds(0, MCH // 2)]],
            gbuf.at[d, pl.ds(0, MCH // 2)], gsem.at[d])

    def _gather_b(j, d):
        return pltpu.make_async_copy(
            y.at[rows2d.at[j, pl.ds(MCH // 2, MCH // 2)]],
            gbuf.at[d, pl.ds(MCH // 2, MCH // 2)], gsem2.at[d])

    def _scatter(j, d):
        return pltpu.make_async_copy(
            gbuf.at[d], acc.at[cols2d.at[j]], ssem.at[d])

    def _phase(p, carry):
        pb = start + p * MPC
        q0 = p * MPC
        pltpu.sync_copy(ei3.at[0, pl.ds(pb, MPC)], rows2d)
        pltpu.sync_copy(ei3.at[1, pl.ds(pb, MPC)], cols2d)

        def _adj(r, cry):
            for v in range(MCH // LANES):
                sl = pl.ds(v * LANES, LANES)
                rows2d[r, sl] = rows2d[r, sl] + off
            return cry

        lax.fori_loop(0, MPC, _adj, 0)

        for j in range(3):
            @pl.when(q0 + j < nch)
            def _(j=j):
                _gather_a(j, j % 4).start()
                _gather_b(j, j % 4).start()

        for j in range(MPC):
            @pl.when(q0 + j < nch)
            def _(j=j):
                d = j % 4
                _gather_a(j, d).wait()
                _gather_b(j, d).wait()
                _scatter(j, d).start(add=True)
                if j + 3 < MPC:
                    @pl.when(q0 + j + 3 < nch)
                    def _():
                        if j >= 1:
                            _scatter(0, (j - 1) % 4).wait()
                        _gather_a(j + 3, (j + 3) % 4).start()
                        _gather_b(j + 3, (j + 3) % 4).start()

        for dd in range(4):
            _scatter(0, dd).wait()
        return carry

    lax.fori_loop(0, MPHASES, _phase, 0)
    plsc.subcore_barrier()

    pltpu.sync_copy(dis.at[pl.ds(n0, NPS)], disv)
    pltpu.sync_copy(b.at[pl.ds(c * HALF, HALF)], bb)

    def _aread(k, d):
        return pltpu.make_async_copy(
            acc.at[pl.ds(n0 + k * MCH, MCH)], gbuf.at[d], gsem.at[d])

    def _owrite_full(k, d):
        return pltpu.make_async_copy(
            gbuf.at[2 + d],
            outp.at[pl.ds(n0 + k * MCH, MCH), pl.ds(c * HALF, HALF)],
            ssem.at[d])

    def _owrite_part(k, d):
        return pltpu.make_async_copy(
            gbuf.at[2 + d, pl.ds(0, N % MCH)],
            outp.at[pl.ds(n0 + k * MCH, N % MCH), pl.ds(c * HALF, HALF)],
            ssem.at[d])

    def _ostart(k, d):
        ws = n0 + k * MCH

        @pl.when(ws + MCH <= N)
        def _():
            _owrite_full(k, d).start()

        @pl.when(jnp.logical_and(ws < N, ws + MCH > N))
        def _():
            _owrite_part(k, d).start()

    def _owait(k, d):
        ws = n0 + k * MCH

        @pl.when(ws + MCH <= N)
        def _():
            _owrite_full(k, d).wait()

        @pl.when(jnp.logical_and(ws < N, ws + MCH > N))
        def _():
            _owrite_part(k, d).wait()

    _aread(0, 0).start()

    def _finpair(k2, carry):
        for d in range(2):
            k = k2 * 2 + d
            _aread(k, d).wait()

            @pl.when(k + 1 < NB)
            def _(d=d, k=k):
                _aread(k + 1, 1 - d).start()

            @pl.when(k >= 2)
            def _(d=d, k=k):
                _owait(k - 2, d)

            def _fin(g, cry, d=d, k=k):
                dvec = disv[pl.ds(k * MCH + g * LANES, LANES)]
                for i in range(LANES):
                    nn = g * LANES + i
                    dval = dvec[i]
                    for v in range(HALF // LANES):
                        sl = pl.ds(v * LANES, LANES)
                        gbuf[2 + d, nn, sl] = jnp.maximum(
                            gbuf[d, nn, sl] * dval + bb[sl], 0.0)
                return cry

            lax.fori_loop(0, MCH // LANES, _fin, 0)
            _ostart(k, d)
        return carry

    lax.fori_loop(0, NB // 2, _finpair, 0)
    _owait(NB - 2, 0)
    _owait(NB - 1, 1)


def kernel(x, edge_index, W, b):
    ei3 = jnp.pad(edge_index,
                  ((0, 0), (0, MSG_CHUNKS * MCH - E))).reshape(
                      2, MSG_CHUNKS, MCH)
    dega = _deg(ei3)
    y3, dis = _mm(x, W, dega.T)
    y = y3.reshape(NC * N_PAD, HALF)
    return _msg(ei3, y, dis.reshape(N_PAD), b)


# speedup vs baseline: 19.7732x; 1.0288x over previous
import functools

import jax
import jax.numpy as jnp
import numpy as np
from jax import lax
from jax.experimental import pallas as pl
from jax.experimental.pallas import tpu as pltpu
from jax.experimental.pallas import tpu_sc as plsc

N = 10000
E = 160000
D_IN = 256
D_OUT = 256
HALF = 128
NC = 2
NS = 16
LANES = 16
N_PAD = 10240
NPS = N_PAD // NS
MCH = 64
MSG_CHUNKS = 2504
DEGCH = 80
MPC = 40
MPHASES = 4
NB = 10
ROW_BLK = 400

_mesh = plsc.VectorSubcoreMesh(
    core_axis_name="c", subcore_axis_name="s", num_cores=NC, num_subcores=NS)


@functools.partial(
    pl.kernel,
    out_type=jax.ShapeDtypeStruct((NC, N_PAD), jnp.float32),
    mesh=_mesh,
    scratch_types=[
        pltpu.VMEM((DEGCH, MCH), jnp.int32),
        pltpu.VMEM((MCH,), jnp.float32),
        pltpu.VMEM((NPS,), jnp.float32),
        pltpu.VMEM_SHARED((N_PAD,), jnp.float32),
        pltpu.SemaphoreType.DMA,
    ],
)
def _deg(ei3, dega, cols2d, onesb, stage, hist, dsem):
    c = lax.axis_index("c")
    s = lax.axis_index("s")
    one_v = jnp.full((LANES,), 1.0, jnp.float32)
    for v in range(MCH // LANES):
        onesb[pl.ds(v * LANES, LANES)] = one_v
    zero_v = jnp.zeros((LANES,), jnp.float32)

    def _z(i, carry):
        stage[pl.ds(i * LANES, LANES)] = zero_v
        return carry

    lax.fori_loop(0, NPS // LANES, _z, 0)
    pltpu.sync_copy(stage, hist.at[pl.ds(s * NPS, NPS)])
    plsc.subcore_barrier()

    w = c * NS + s
    ten = w < 24
    start = jnp.where(ten, 80 * w, 1920 + 72 * (w - 24))
    nch = jnp.where(w == 31, 76, jnp.where(ten, 80, 72))

    @pl.when(ten)
    def _():
        pltpu.sync_copy(ei3.at[1, pl.ds(start, 80)], cols2d.at[pl.ds(0, 80)])

    @pl.when(jnp.logical_not(ten))
    def _():
        pltpu.sync_copy(ei3.at[1, pl.ds(start, 72)], cols2d.at[pl.ds(0, 72)])

    @pl.when(w == 31)
    def _():
        pltpu.sync_copy(ei3.at[1, pl.ds(2496, 4)], cols2d.at[pl.ds(72, 4)])

    for j in range(DEGCH):
        @pl.when(j < nch)
        def _(j=j):
            pltpu.make_async_copy(
                onesb, hist.at[cols2d.at[j]], dsem).start(add=True)
    for j in range(DEGCH):
        @pl.when(j < nch)
        def _(j=j):
            pltpu.make_async_copy(onesb, hist.at[cols2d.at[0]], dsem).wait()

    plsc.subcore_barrier()
    pltpu.sync_copy(hist.at[pl.ds(s * NPS, NPS)], stage)
    pltpu.sync_copy(stage, dega.at[c, pl.ds(s * NPS, NPS)])


def _mm_body(x_ref, w_ref, degt_ref, y_ref, dis_ref):
    deg = degt_ref[:, 0:1] + degt_ref[:, 1:2] + 1.0
    dis = lax.rsqrt(deg)
    xw = jnp.dot(x_ref[...], w_ref[...],
                 preferred_element_type=jnp.float32)
    y_ref[0] = xw * dis
    dis_ref[...] = dis


_mm = pl.pallas_call(
    _mm_body,
    grid=(NC, N // ROW_BLK),
    in_specs=[
        pl.BlockSpec((ROW_BLK, D_IN), lambda c, i: (i, 0)),
        pl.BlockSpec((D_IN, HALF), lambda c, i: (0, c)),
        pl.BlockSpec((ROW_BLK, 2), lambda c, i: (i, 0)),
    ],
    out_specs=[
        pl.BlockSpec((1, ROW_BLK, HALF), lambda c, i: (c, i, 0)),
        pl.BlockSpec((ROW_BLK, 1), lambda c, i: (i, 0)),
    ],
    out_shape=[
        jax.ShapeDtypeStruct((NC, N_PAD, HALF), jnp.float32),
        jax.ShapeDtypeStruct((N_PAD, 1), jnp.float32),
    ],
)


@functools.partial(
    pl.kernel,
    out_type=jax.ShapeDtypeStruct((N, D_OUT), jnp.float32),
    mesh=_mesh,
    scratch_types=[
        pltpu.VMEM((MPC, MCH), jnp.int32),
        pltpu.VMEM((MPC, MCH), jnp.int32),
        pltpu.VMEM((4, MCH, HALF), jnp.float32),
        pltpu.VMEM((NPS,), jnp.float32),
        pltpu.VMEM((HALF,), jnp.float32),
        pltpu.VMEM_SHARED((N_PAD, HALF), jnp.float32),
        pltpu.SemaphoreType.DMA((4,)),
        pltpu.SemaphoreType.DMA((4,)),
        pltpu.SemaphoreType.DMA((4,)),
    ],
)
def _msg(ei3, y, dis, b, outp, rows2d, cols2d, gbuf, disv, bb, acc,
         gsem, gsem2, ssem):
    c = lax.axis_index("c")
    s = lax.axis_index("s")
    n0 = s * NPS

    def _yload(k, d):
        return pltpu.make_async_copy(
            y.at[pl.ds(c * N_PAD + n0 + k * MCH, MCH)], gbuf.at[d],
            gsem.at[d])

    _yload(0, 0).start()
    _yload(1, 1).start()
    for k in range(NB):
        d = k % 2
        _yload(k, d).wait()
        pltpu.sync_copy(gbuf.at[d], acc.at[pl.ds(n0 + k * MCH, MCH)])
        if k + 2 < NB:
            _yload(k + 2, d).start()

    nch = jnp.where(s < 8, 160, jnp.where(s < 15, 152, 156))
    start = jnp.where(s < 8, 160 * s, 1280 + 152 * (s - 8))
    off = c * N_PAD
    plsc.subcore_barrier()

    def _gather_a(j, d):
        return pltpu.make_async_copy(
            y.at[rows2d.at[j, pl.ds(0, MCH // 2)]],
            gbuf.at[d, pl.ds(0, MCH // 2)], gsem.at[d])

    def _gather_b(j, d):
        return pltpu.make_async_copy(
            y.at[rows2d.at[j, pl.ds(MCH // 2, MCH // 2)]],
            gbuf.at[d, pl.ds(MCH // 2, MCH // 2)], gsem2.at[d])

    def _scatter(j, d):
        return pltpu.make_async_copy(
            gbuf.at[d], acc.at[cols2d.at[j]], ssem.at[d])

    def _phase(p, carry):
        pb = start + p * MPC
        q0 = p * MPC
        pltpu.sync_copy(ei3.at[0, pl.ds(pb, MPC)], rows2d)
        pltpu.sync_copy(ei3.at[1, pl.ds(pb, MPC)], cols2d)

        def _adj(r, cry):
            for v in range(MCH // LANES):
                sl = pl.ds(v * LANES, LANES)
                rows2d[r, sl] = rows2d[r, sl] + off
            return cry

        lax.fori_loop(0, MPC, _adj, 0)

        for j in range(3):
            @pl.when(q0 + j < nch)
            def _(j=j):
                _gather_a(j, j % 4).start()
                _gather_b(j, j % 4).start()

        for j in range(MPC):
            @pl.when(q0 + j < nch)
            def _(j=j):
                d = j % 4
                _gather_a(j, d).wait()
                _gather_b(j, d).wait()
                _scatter(j, d).start(add=True)
                if j + 3 < MPC:
                    @pl.when(q0 + j + 3 < nch)
                    def _():
                        if j >= 1:
                            _scatter(0, (j - 1) % 4).wait()
                        _gather_a(j + 3, (j + 3) % 4).start()
                        _gather_b(j + 3, (j + 3) % 4).start()

        for dd in range(4):
            _scatter(0, dd).wait()
        return carry

    lax.fori_loop(0, MPHASES, _phase, 0)
    plsc.subcore_barrier()

    pltpu.sync_copy(dis.at[pl.ds(n0, NPS)], disv)
    pltpu.sync_copy(b.at[pl.ds(c * HALF, HALF)], bb)

    def _aread(k, d):
        return pltpu.make_async_copy(
            acc.at[pl.ds(n0 + k * MCH, MCH)], gbuf.at[d], gsem.at[d])

    def _owrite_full(k, d):
        return pltpu.make_async_copy(
            gbuf.at[2 + d],
            outp.at[pl.ds(n0 + k * MCH, MCH), pl.ds(c * HALF, HALF)],
            ssem.at[d])

    def _owrite_part(k, d):
        return pltpu.make_async_copy(
            gbuf.at[2 + d, pl.ds(0, N % MCH)],
            outp.at[pl.ds(n0 + k * MCH, N % MCH), pl.ds(c * HALF, HALF)],
            ssem.at[d])

    def _ostart(k, d):
        ws = n0 + k * MCH

        @pl.when(ws + MCH <= N)
        def _():
            _owrite_full(k, d).start()

        @pl.when(jnp.logical_and(ws < N, ws + MCH > N))
        def _():
            _owrite_part(k, d).start()

    def _owait(k, d):
        ws = n0 + k * MCH

        @pl.when(ws + MCH <= N)
        def _():
            _owrite_full(k, d).wait()

        @pl.when(jnp.logical_and(ws < N, ws + MCH > N))
        def _():
            _owrite_part(k, d).wait()

    _aread(0, 0).start()

    def _finpair(k2, carry):
        for d in range(2):
            k = k2 * 2 + d
            _aread(k, d).wait()

            @pl.when(k + 1 < NB)
            def _(d=d, k=k):
                _aread(k + 1, 1 - d).start()

            @pl.when(k >= 2)
            def _(d=d, k=k):
                _owait(k - 2, d)

            def _fin(g, cry, d=d, k=k):
                dvec = disv[pl.ds(k * MCH + g * LANES, LANES)]
                for i in range(LANES):
                    nn = g * LANES + i
                    dval = dvec[i]
                    for v in range(HALF // LANES):
                        sl = pl.ds(v * LANES, LANES)
                        gbuf[2 + d, nn, sl] = jnp.maximum(
                            gbuf[d, nn, sl] * dval + bb[sl], 0.0)
                return cry

            lax.fori_loop(0, MCH // LANES, _fin, 0)
            _ostart(k, d)
        return carry

    lax.fori_loop(0, NB // 2, _finpair, 0)
    _owait(NB - 2, 0)
    _owait(NB - 1, 1)


def kernel(x, edge_index, W, b):
    ei3 = jnp.pad(edge_index,
                  ((0, 0), (0, MSG_CHUNKS * MCH - E))).reshape(
                      2, MSG_CHUNKS, MCH)
    dega = _deg(ei3)
    y3, dis = _mm(x, W, dega.T)
    y = y3.reshape(NC * N_PAD, HALF)
    return _msg(ei3, y, dis.reshape(N_PAD), b)


# speedup vs baseline: 21.7404x; 1.0995x over previous
import functools

import jax
import jax.numpy as jnp
import numpy as np
from jax import lax
from jax.experimental import pallas as pl
from jax.experimental.pallas import tpu as pltpu
from jax.experimental.pallas import tpu_sc as plsc

N = 10000
E = 160000
D_IN = 256
D_OUT = 256
HALF = 128
NC = 2
NS = 16
LANES = 16
N_PAD = 10240
NPS = N_PAD // NS
MCH = 64
MSG_CHUNKS = 2504
DEGCH = 80
MPC = 40
MPHASES = 4
NB = 10
ROW_BLK = 400

_mesh = plsc.VectorSubcoreMesh(
    core_axis_name="c", subcore_axis_name="s", num_cores=NC, num_subcores=NS)


@functools.partial(
    pl.kernel,
    out_type=jax.ShapeDtypeStruct((NC, N_PAD), jnp.float32),
    mesh=_mesh,
    scratch_types=[
        pltpu.VMEM((DEGCH, MCH), jnp.int32),
        pltpu.VMEM((MCH,), jnp.float32),
        pltpu.VMEM((NPS,), jnp.float32),
        pltpu.VMEM_SHARED((N_PAD,), jnp.float32),
        pltpu.SemaphoreType.DMA,
    ],
)
def _deg(ei3, dega, cols2d, onesb, stage, hist, dsem):
    c = lax.axis_index("c")
    s = lax.axis_index("s")
    one_v = jnp.full((LANES,), 1.0, jnp.float32)
    for v in range(MCH // LANES):
        onesb[pl.ds(v * LANES, LANES)] = one_v
    zero_v = jnp.zeros((LANES,), jnp.float32)

    def _z(i, carry):
        stage[pl.ds(i * LANES, LANES)] = zero_v
        return carry

    lax.fori_loop(0, NPS // LANES, _z, 0)
    pltpu.sync_copy(stage, hist.at[pl.ds(s * NPS, NPS)])
    plsc.subcore_barrier()

    w = c * NS + s
    ten = w < 24
    start = jnp.where(ten, 80 * w, 1920 + 72 * (w - 24))
    nch = jnp.where(w == 31, 76, jnp.where(ten, 80, 72))

    @pl.when(ten)
    def _():
        pltpu.sync_copy(ei3.at[1, pl.ds(start, 80)], cols2d.at[pl.ds(0, 80)])

    @pl.when(jnp.logical_not(ten))
    def _():
        pltpu.sync_copy(ei3.at[1, pl.ds(start, 72)], cols2d.at[pl.ds(0, 72)])

    @pl.when(w == 31)
    def _():
        pltpu.sync_copy(ei3.at[1, pl.ds(2496, 4)], cols2d.at[pl.ds(72, 4)])

    for j in range(DEGCH):
        @pl.when(j < nch)
        def _(j=j):
            pltpu.make_async_copy(
                onesb, hist.at[cols2d.at[j]], dsem).start(add=True)
    for j in range(DEGCH):
        @pl.when(j < nch)
        def _(j=j):
            pltpu.make_async_copy(onesb, hist.at[cols2d.at[0]], dsem).wait()

    plsc.subcore_barrier()
    pltpu.sync_copy(hist.at[pl.ds(s * NPS, NPS)], stage)
    pltpu.sync_copy(stage, dega.at[c, pl.ds(s * NPS, NPS)])


def _mm_body(x_ref, w_ref, degt_ref, y_ref, dis_ref):
    deg = degt_ref[:, 0:1] + degt_ref[:, 1:2] + 1.0
    dis = lax.rsqrt(deg)
    xw = jnp.dot(x_ref[...], w_ref[...],
                 preferred_element_type=jnp.float32)
    y_ref[0] = xw[:, :HALF] * dis
    y_ref[1] = xw[:, HALF:] * dis
    dis_ref[...] = dis


_mm = pl.pallas_call(
    _mm_body,
    grid=(N // ROW_BLK,),
    in_specs=[
        pl.BlockSpec((ROW_BLK, D_IN), lambda i: (i, 0)),
        pl.BlockSpec((D_IN, D_OUT), lambda i: (0, 0)),
        pl.BlockSpec((ROW_BLK, 2), lambda i: (i, 0)),
    ],
    out_specs=[
        pl.BlockSpec((NC, ROW_BLK, HALF), lambda i: (0, i, 0)),
        pl.BlockSpec((ROW_BLK, 1), lambda i: (i, 0)),
    ],
    out_shape=[
        jax.ShapeDtypeStruct((NC, N_PAD, HALF), jnp.float32),
        jax.ShapeDtypeStruct((N_PAD, 1), jnp.float32),
    ],
)


@functools.partial(
    pl.kernel,
    out_type=jax.ShapeDtypeStruct((N, D_OUT), jnp.float32),
    mesh=_mesh,
    scratch_types=[
        pltpu.VMEM((MPC, MCH), jnp.int32),
        pltpu.VMEM((MPC, MCH), jnp.int32),
        pltpu.VMEM((4, MCH, HALF), jnp.float32),
        pltpu.VMEM((NPS,), jnp.float32),
        pltpu.VMEM((HALF,), jnp.float32),
        pltpu.VMEM_SHARED((N_PAD, HALF), jnp.float32),
        pltpu.SemaphoreType.DMA((4,)),
        pltpu.SemaphoreType.DMA((4,)),
        pltpu.SemaphoreType.DMA((4,)),
    ],
)
def _msg(ei3, y, dis, b, outp, rows2d, cols2d, gbuf, disv, bb, acc,
         gsem, gsem2, ssem):
    c = lax.axis_index("c")
    s = lax.axis_index("s")
    n0 = s * NPS

    def _yload(k, d):
        return pltpu.make_async_copy(
            y.at[pl.ds(c * N_PAD + n0 + k * MCH, MCH)], gbuf.at[d],
            gsem.at[d])

    _yload(0, 0).start()
    _yload(1, 1).start()
    for k in range(NB):
        d = k % 2
        _yload(k, d).wait()
        pltpu.sync_copy(gbuf.at[d], acc.at[pl.ds(n0 + k * MCH, MCH)])
        if k + 2 < NB:
            _yload(k + 2, d).start()

    nch = jnp.where(s < 8, 160, jnp.where(s < 15, 152, 156))
    start = jnp.where(s < 8, 160 * s, 1280 + 152 * (s - 8))
    off = c * N_PAD
    plsc.subcore_barrier()

    def _gather_a(j, d):
        return pltpu.make_async_copy(
            y.at[rows2d.at[j, pl.ds(0, MCH // 2)]],
            gbuf.at[d, pl.ds(0, MCH // 2)], gsem.at[d])

    def _gather_b(j, d):
        return pltpu.make_async_copy(
            y.at[rows2d.at[j, pl.ds(MCH // 2, MCH // 2)]],
            gbuf.at[d, pl.ds(MCH // 2, MCH // 2)], gsem2.at[d])

    def _scatter(j, d):
        return pltpu.make_async_copy(
            gbuf.at[d], acc.at[cols2d.at[j]], ssem.at[d])

    def _phase(p, carry):
        pb = start + p * MPC
        q0 = p * MPC
        pltpu.sync_copy(ei3.at[0, pl.ds(pb, MPC)], rows2d)
        pltpu.sync_copy(ei3.at[1, pl.ds(pb, MPC)], cols2d)

        def _adj(r, cry):
            for v in range(MCH // LANES):
                sl = pl.ds(v * LANES, LANES)
                rows2d[r, sl] = rows2d[r, sl] + off
            return cry

        lax.fori_loop(0, MPC, _adj, 0)

        for j in range(3):
            @pl.when(q0 + j < nch)
            def _(j=j):
                _gather_a(j, j % 4).start()
                _gather_b(j, j % 4).start()

        for j in range(MPC):
            @pl.when(q0 + j < nch)
            def _(j=j):
                d = j % 4
                _gather_a(j, d).wait()
                _gather_b(j, d).wait()
                _scatter(j, d).start(add=True)
                if j + 3 < MPC:
                    @pl.when(q0 + j + 3 < nch)
                    def _():
                        if j >= 1:
                            _scatter(0, (j - 1) % 4).wait()
                        _gather_a(j + 3, (j + 3) % 4).start()
                        _gather_b(j + 3, (j + 3) % 4).start()

        for dd in range(4):
            _scatter(0, dd).wait()
        return carry

    lax.fori_loop(0, MPHASES, _phase, 0)
    plsc.subcore_barrier()

    pltpu.sync_copy(dis.at[pl.ds(n0, NPS)], disv)
    pltpu.sync_copy(b.at[pl.ds(c * HALF, HALF)], bb)

    def _aread(k, d):
        return pltpu.make_async_copy(
            acc.at[pl.ds(n0 + k * MCH, MCH)], gbuf.at[d], gsem.at[d])

    def _owrite_full(k, d):
        return pltpu.make_async_copy(
            gbuf.at[2 + d],
            outp.at[pl.ds(n0 + k * MCH, MCH), pl.ds(c * HALF, HALF)],
            ssem.at[d])

    def _owrite_part(k, d):
        return pltpu.make_async_copy(
            gbuf.at[2 + d, pl.ds(0, N % MCH)],
            outp.at[pl.ds(n0 + k * MCH, N % MCH), pl.ds(c * HALF, HALF)],
            ssem.at[d])

    def _ostart(k, d):
        ws = n0 + k * MCH

        @pl.when(ws + MCH <= N)
        def _():
            _owrite_full(k, d).start()

        @pl.when(jnp.logical_and(ws < N, ws + MCH > N))
        def _():
            _owrite_part(k, d).start()

    def _owait(k, d):
        ws = n0 + k * MCH

        @pl.when(ws + MCH <= N)
        def _():
            _owrite_full(k, d).wait()

        @pl.when(jnp.logical_and(ws < N, ws + MCH > N))
        def _():
            _owrite_part(k, d).wait()

    _aread(0, 0).start()

    def _finpair(k2, carry):
        for d in range(2):
            k = k2 * 2 + d
            _aread(k, d).wait()

            @pl.when(k + 1 < NB)
            def _(d=d, k=k):
                _aread(k + 1, 1 - d).start()

            @pl.when(k >= 2)
            def _(d=d, k=k):
                _owait(k - 2, d)

            def _fin(g, cry, d=d, k=k):
                dvec = disv[pl.ds(k * MCH + g * LANES, LANES)]
                for i in range(LANES):
                    nn = g * LANES + i
                    dval = dvec[i]
                    for v in range(HALF // LANES):
                        sl = pl.ds(v * LANES, LANES)
                        gbuf[2 + d, nn, sl] = jnp.maximum(
                            gbuf[d, nn, sl] * dval + bb[sl], 0.0)
                return cry

            lax.fori_loop(0, MCH // LANES, _fin, 0)
            _ostart(k, d)
        return carry

    lax.fori_loop(0, NB // 2, _finpair, 0)
    _owait(NB - 2, 0)
    _owait(NB - 1, 1)


def kernel(x, edge_index, W, b):
    ei3 = jnp.pad(edge_index,
                  ((0, 0), (0, MSG_CHUNKS * MCH - E))).reshape(
                      2, MSG_CHUNKS, MCH)
    dega = _deg(ei3)
    y3, dis = _mm(x, W, dega.T)
    y = y3.reshape(NC * N_PAD, HALF)
    return _msg(ei3, y, dis.reshape(N_PAD), b)


# speedup vs baseline: 21.7563x; 1.0007x over previous
import functools

import jax
import jax.numpy as jnp
import numpy as np
from jax import lax
from jax.experimental import pallas as pl
from jax.experimental.pallas import tpu as pltpu
from jax.experimental.pallas import tpu_sc as plsc

N = 10000
E = 160000
D_IN = 256
D_OUT = 256
HALF = 128
NC = 2
NS = 16
LANES = 16
N_PAD = 10240
NPS = N_PAD // NS
MCH = 64
MSG_CHUNKS = 2500
DEGCH = 80
MPC = 40
MPHASES = 4
NB = 10
ROW_BLK = 400

_mesh = plsc.VectorSubcoreMesh(
    core_axis_name="c", subcore_axis_name="s", num_cores=NC, num_subcores=NS)


@functools.partial(
    pl.kernel,
    out_type=jax.ShapeDtypeStruct((NC, N_PAD), jnp.float32),
    mesh=_mesh,
    scratch_types=[
        pltpu.VMEM((DEGCH, MCH), jnp.int32),
        pltpu.VMEM((MCH,), jnp.float32),
        pltpu.VMEM((NPS,), jnp.float32),
        pltpu.VMEM_SHARED((N_PAD,), jnp.float32),
        pltpu.SemaphoreType.DMA,
    ],
)
def _deg(ei3, dega, cols2d, onesb, stage, hist, dsem):
    c = lax.axis_index("c")
    s = lax.axis_index("s")
    one_v = jnp.full((LANES,), 1.0, jnp.float32)
    for v in range(MCH // LANES):
        onesb[pl.ds(v * LANES, LANES)] = one_v
    zero_v = jnp.zeros((LANES,), jnp.float32)

    def _z(i, carry):
        stage[pl.ds(i * LANES, LANES)] = zero_v
        return carry

    lax.fori_loop(0, NPS // LANES, _z, 0)
    pltpu.sync_copy(stage, hist.at[pl.ds(s * NPS, NPS)])
    plsc.subcore_barrier()

    w = c * NS + s
    ten = w < 24
    start = jnp.where(ten, 80 * w, 1920 + 72 * (w - 24))
    nch = jnp.where(w == 31, 76, jnp.where(ten, 80, 72))

    @pl.when(ten)
    def _():
        pltpu.sync_copy(ei3.at[1, pl.ds(start, 80)], cols2d.at[pl.ds(0, 80)])

    @pl.when(jnp.logical_not(ten))
    def _():
        pltpu.sync_copy(ei3.at[1, pl.ds(start, 72)], cols2d.at[pl.ds(0, 72)])

    @pl.when(w == 31)
    def _():
        pltpu.sync_copy(ei3.at[1, pl.ds(2496, 4)], cols2d.at[pl.ds(72, 4)])

    for j in range(DEGCH):
        @pl.when(j < nch)
        def _(j=j):
            pltpu.make_async_copy(
                onesb, hist.at[cols2d.at[j]], dsem).start(add=True)
    for j in range(DEGCH):
        @pl.when(j < nch)
        def _(j=j):
            pltpu.make_async_copy(onesb, hist.at[cols2d.at[0]], dsem).wait()

    plsc.subcore_barrier()
    pltpu.sync_copy(hist.at[pl.ds(s * NPS, NPS)], stage)
    pltpu.sync_copy(stage, dega.at[c, pl.ds(s * NPS, NPS)])


def _mm_body(x_ref, w_ref, degt_ref, y_ref, dis_ref):
    deg = degt_ref[:, 0:1] + degt_ref[:, 1:2] + 1.0
    dis = lax.rsqrt(deg)
    xw = jnp.dot(x_ref[...], w_ref[...],
                 preferred_element_type=jnp.float32)
    y_ref[0] = xw[:, :HALF] * dis
    y_ref[1] = xw[:, HALF:] * dis
    dis_ref[...] = dis


_mm = pl.pallas_call(
    _mm_body,
    grid=(N // ROW_BLK,),
    in_specs=[
        pl.BlockSpec((ROW_BLK, D_IN), lambda i: (i, 0)),
        pl.BlockSpec((D_IN, D_OUT), lambda i: (0, 0)),
        pl.BlockSpec((ROW_BLK, 2), lambda i: (i, 0)),
    ],
    out_specs=[
        pl.BlockSpec((NC, ROW_BLK, HALF), lambda i: (0, i, 0)),
        pl.BlockSpec((ROW_BLK, 1), lambda i: (i, 0)),
    ],
    out_shape=[
        jax.ShapeDtypeStruct((NC, N_PAD, HALF), jnp.float32),
        jax.ShapeDtypeStruct((N_PAD, 1), jnp.float32),
    ],
)


@functools.partial(
    pl.kernel,
    out_type=jax.ShapeDtypeStruct((N, D_OUT), jnp.float32),
    mesh=_mesh,
    scratch_types=[
        pltpu.VMEM((MPC, MCH), jnp.int32),
        pltpu.VMEM((MPC, MCH), jnp.int32),
        pltpu.VMEM((4, MCH, HALF), jnp.float32),
        pltpu.VMEM((NPS,), jnp.float32),
        pltpu.VMEM((HALF,), jnp.float32),
        pltpu.VMEM_SHARED((N_PAD, HALF), jnp.float32),
        pltpu.SemaphoreType.DMA((4,)),
        pltpu.SemaphoreType.DMA((4,)),
        pltpu.SemaphoreType.DMA((4,)),
    ],
)
def _msg(ei3, y, dis, b, outp, rows2d, cols2d, gbuf, disv, bb, acc,
         gsem, gsem2, ssem):
    c = lax.axis_index("c")
    s = lax.axis_index("s")
    n0 = s * NPS

    def _yload(k, d):
        return pltpu.make_async_copy(
            y.at[pl.ds(c * N_PAD + n0 + k * MCH, MCH)], gbuf.at[d],
            gsem.at[d])

    _yload(0, 0).start()
    _yload(1, 1).start()
    for k in range(NB):
        d = k % 2
        _yload(k, d).wait()
        pltpu.sync_copy(gbuf.at[d], acc.at[pl.ds(n0 + k * MCH, MCH)])
        if k + 2 < NB:
            _yload(k + 2, d).start()

    nch = jnp.where(s < 8, 160, jnp.where(s < 15, 152, 156))
    start = jnp.where(s < 8, 160 * s, 1280 + 152 * (s - 8))
    off = c * N_PAD
    plsc.subcore_barrier()

    def _gather_a(j, d):
        return pltpu.make_async_copy(
            y.at[rows2d.at[j, pl.ds(0, MCH // 2)]],
            gbuf.at[d, pl.ds(0, MCH // 2)], gsem.at[d])

    def _gather_b(j, d):
        return pltpu.make_async_copy(
            y.at[rows2d.at[j, pl.ds(MCH // 2, MCH // 2)]],
            gbuf.at[d, pl.ds(MCH // 2, MCH // 2)], gsem2.at[d])

    def _scatter(j, d):
        return pltpu.make_async_copy(
            gbuf.at[d], acc.at[cols2d.at[j]], ssem.at[d])

    def _phase(p, carry):
        pb = start + p * MPC
        q0 = p * MPC
        last15 = jnp.logical_and(s == 15, p == MPHASES - 1)

        @pl.when(jnp.logical_not(last15))
        def _():
            pltpu.sync_copy(ei3.at[0, pl.ds(pb, MPC)], rows2d)
            pltpu.sync_copy(ei3.at[1, pl.ds(pb, MPC)], cols2d)

        @pl.when(last15)
        def _():
            pltpu.sync_copy(ei3.at[0, pl.ds(pb, 32)], rows2d.at[pl.ds(0, 32)])
            pltpu.sync_copy(ei3.at[1, pl.ds(pb, 32)], cols2d.at[pl.ds(0, 32)])
            pltpu.sync_copy(ei3.at[0, pl.ds(2496, 4)],
                            rows2d.at[pl.ds(32, 4)])
            pltpu.sync_copy(ei3.at[1, pl.ds(2496, 4)],
                            cols2d.at[pl.ds(32, 4)])

        def _adj(r, cry):
            for v in range(MCH // LANES):
                sl = pl.ds(v * LANES, LANES)
                rows2d[r, sl] = rows2d[r, sl] + off
            return cry

        lax.fori_loop(0, MPC, _adj, 0)

        for j in range(3):
            @pl.when(q0 + j < nch)
            def _(j=j):
                _gather_a(j, j % 4).start()
                _gather_b(j, j % 4).start()

        for j in range(MPC):
            @pl.when(q0 + j < nch)
            def _(j=j):
                d = j % 4
                _gather_a(j, d).wait()
                _gather_b(j, d).wait()
                _scatter(j, d).start(add=True)
                if j + 3 < MPC:
                    @pl.when(q0 + j + 3 < nch)
                    def _():
                        if j >= 1:
                            _scatter(0, (j - 1) % 4).wait()
                        _gather_a(j + 3, (j + 3) % 4).start()
                        _gather_b(j + 3, (j + 3) % 4).start()

        for dd in range(4):
            _scatter(0, dd).wait()
        return carry

    lax.fori_loop(0, MPHASES, _phase, 0)
    plsc.subcore_barrier()

    pltpu.sync_copy(dis.at[pl.ds(n0, NPS)], disv)
    pltpu.sync_copy(b.at[pl.ds(c * HALF, HALF)], bb)

    def _aread(k, d):
        return pltpu.make_async_copy(
            acc.at[pl.ds(n0 + k * MCH, MCH)], gbuf.at[d], gsem.at[d])

    def _owrite_full(k, d):
        return pltpu.make_async_copy(
            gbuf.at[2 + d],
            outp.at[pl.ds(n0 + k * MCH, MCH), pl.ds(c * HALF, HALF)],
            ssem.at[d])

    def _owrite_part(k, d):
        return pltpu.make_async_copy(
            gbuf.at[2 + d, pl.ds(0, N % MCH)],
            outp.at[pl.ds(n0 + k * MCH, N % MCH), pl.ds(c * HALF, HALF)],
            ssem.at[d])

    def _ostart(k, d):
        ws = n0 + k * MCH

        @pl.when(ws + MCH <= N)
        def _():
            _owrite_full(k, d).start()

        @pl.when(jnp.logical_and(ws < N, ws + MCH > N))
        def _():
            _owrite_part(k, d).start()

    def _owait(k, d):
        ws = n0 + k * MCH

        @pl.when(ws + MCH <= N)
        def _():
            _owrite_full(k, d).wait()

        @pl.when(jnp.logical_and(ws < N, ws + MCH > N))
        def _():
            _owrite_part(k, d).wait()

    _aread(0, 0).start()

    def _finpair(k2, carry):
        for d in range(2):
            k = k2 * 2 + d
            _aread(k, d).wait()

            @pl.when(k + 1 < NB)
            def _(d=d, k=k):
                _aread(k + 1, 1 - d).start()

            @pl.when(k >= 2)
            def _(d=d, k=k):
                _owait(k - 2, d)

            def _fin(g, cry, d=d, k=k):
                dvec = disv[pl.ds(k * MCH + g * LANES, LANES)]
                for i in range(LANES):
                    nn = g * LANES + i
                    dval = dvec[i]
                    for v in range(HALF // LANES):
                        sl = pl.ds(v * LANES, LANES)
                        gbuf[2 + d, nn, sl] = jnp.maximum(
                            gbuf[d, nn, sl] * dval + bb[sl], 0.0)
                return cry

            lax.fori_loop(0, MCH // LANES, _fin, 0)
            _ostart(k, d)
        return carry

    lax.fori_loop(0, NB // 2, _finpair, 0)
    _owait(NB - 2, 0)
    _owait(NB - 1, 1)


def kernel(x, edge_index, W, b):
    ei3 = edge_index.reshape(2, MSG_CHUNKS, MCH)
    dega = _deg(ei3)
    y3, dis = _mm(x, W, dega.T)
    y = y3.reshape(NC * N_PAD, HALF)
    return _msg(ei3, y, dis.reshape(N_PAD), b)


# speedup vs baseline: 21.7643x; 1.0004x over previous
import functools

import jax
import jax.numpy as jnp
from jax import lax
from jax.experimental import pallas as pl
from jax.experimental.pallas import tpu as pltpu
from jax.experimental.pallas import tpu_sc as plsc

N = 10000
E = 160000
D_IN = 256
D_OUT = 256
HALF = 128
NC = 2
NS = 16
LANES = 16
N_PAD = 10240
NPS = N_PAD // NS
MCH = 64
MSG_CHUNKS = 2500
DEGCH = 80
MPC = 40
MPHASES = 4
NB = 10
ROW_BLK = 400

_mesh = plsc.VectorSubcoreMesh(
    core_axis_name="c", subcore_axis_name="s", num_cores=NC, num_subcores=NS)


@functools.partial(
    pl.kernel,
    out_type=jax.ShapeDtypeStruct((NC, N_PAD), jnp.float32),
    mesh=_mesh,
    scratch_types=[
        pltpu.VMEM((DEGCH, MCH), jnp.int32),
        pltpu.VMEM((MCH,), jnp.float32),
        pltpu.VMEM((NPS,), jnp.float32),
        pltpu.VMEM_SHARED((N_PAD,), jnp.float32),
        pltpu.SemaphoreType.DMA,
    ],
)
def _deg(ei3, dega, cols2d, onesb, stage, hist, dsem):
    c = lax.axis_index("c")
    s = lax.axis_index("s")
    one_v = jnp.full((LANES,), 1.0, jnp.float32)
    for v in range(MCH // LANES):
        onesb[pl.ds(v * LANES, LANES)] = one_v
    zero_v = jnp.zeros((LANES,), jnp.float32)

    def _z(i, carry):
        stage[pl.ds(i * LANES, LANES)] = zero_v
        return carry

    lax.fori_loop(0, NPS // LANES, _z, 0)
    pltpu.sync_copy(stage, hist.at[pl.ds(s * NPS, NPS)])
    plsc.subcore_barrier()

    w = c * NS + s
    ten = w < 24
    start = jnp.where(ten, 80 * w, 1920 + 72 * (w - 24))
    nch = jnp.where(w == 31, 76, jnp.where(ten, 80, 72))

    @pl.when(ten)
    def _():
        pltpu.sync_copy(ei3.at[1, pl.ds(start, 80)], cols2d.at[pl.ds(0, 80)])

    @pl.when(jnp.logical_not(ten))
    def _():
        pltpu.sync_copy(ei3.at[1, pl.ds(start, 72)], cols2d.at[pl.ds(0, 72)])

    @pl.when(w == 31)
    def _():
        pltpu.sync_copy(ei3.at[1, pl.ds(2496, 4)], cols2d.at[pl.ds(72, 4)])

    for j in range(DEGCH):
        @pl.when(j < nch)
        def _(j=j):
            pltpu.make_async_copy(
                onesb, hist.at[cols2d.at[j]], dsem).start(add=True)
    for j in range(DEGCH):
        @pl.when(j < nch)
        def _(j=j):
            pltpu.make_async_copy(onesb, hist.at[cols2d.at[0]], dsem).wait()

    plsc.subcore_barrier()
    pltpu.sync_copy(hist.at[pl.ds(s * NPS, NPS)], stage)
    pltpu.sync_copy(stage, dega.at[c, pl.ds(s * NPS, NPS)])


def _mm_body(x_ref, w_ref, degt_ref, y_ref, dis_ref):
    deg = degt_ref[:, 0:1] + degt_ref[:, 1:2] + 1.0
    dis = lax.rsqrt(deg)
    xw = jnp.dot(x_ref[...], w_ref[...],
                 preferred_element_type=jnp.float32)
    y_ref[0] = xw[:, :HALF] * dis
    y_ref[1] = xw[:, HALF:] * dis
    dis_ref[...] = dis


_mm = pl.pallas_call(
    _mm_body,
    grid=(N // ROW_BLK,),
    in_specs=[
        pl.BlockSpec((ROW_BLK, D_IN), lambda i: (i, 0)),
        pl.BlockSpec((D_IN, D_OUT), lambda i: (0, 0)),
        pl.BlockSpec((ROW_BLK, 2), lambda i: (i, 0)),
    ],
    out_specs=[
        pl.BlockSpec((NC, ROW_BLK, HALF), lambda i: (0, i, 0)),
        pl.BlockSpec((ROW_BLK, 1), lambda i: (i, 0)),
    ],
    out_shape=[
        jax.ShapeDtypeStruct((NC, N_PAD, HALF), jnp.float32),
        jax.ShapeDtypeStruct((N_PAD, 1), jnp.float32),
    ],
)


@functools.partial(
    pl.kernel,
    out_type=jax.ShapeDtypeStruct((N, D_OUT), jnp.float32),
    mesh=_mesh,
    scratch_types=[
        pltpu.VMEM((MPC, MCH), jnp.int32),
        pltpu.VMEM((MPC, MCH), jnp.int32),
        pltpu.VMEM((4, MCH, HALF), jnp.float32),
        pltpu.VMEM((NPS,), jnp.float32),
        pltpu.VMEM((HALF,), jnp.float32),
        pltpu.VMEM_SHARED((N_PAD, HALF), jnp.float32),
        pltpu.SemaphoreType.DMA((4,)),
        pltpu.SemaphoreType.DMA((4,)),
        pltpu.SemaphoreType.DMA((4,)),
    ],
)
def _msg(ei3, y, dis, b, outp, rows2d, cols2d, gbuf, disv, bb, acc,
         gsem, gsem2, ssem):
    c = lax.axis_index("c")
    s = lax.axis_index("s")
    n0 = s * NPS

    def _yload(k, d):
        return pltpu.make_async_copy(
            y.at[pl.ds(c * N_PAD + n0 + k * MCH, MCH)], gbuf.at[d],
            gsem.at[d])

    _yload(0, 0).start()
    _yload(1, 1).start()
    for k in range(NB):
        d = k % 2
        _yload(k, d).wait()
        pltpu.sync_copy(gbuf.at[d], acc.at[pl.ds(n0 + k * MCH, MCH)])
        if k + 2 < NB:
            _yload(k + 2, d).start()

    nch = jnp.where(s < 8, 160, jnp.where(s < 15, 152, 156))
    start = jnp.where(s < 8, 160 * s, 1280 + 152 * (s - 8))
    off = c * N_PAD
    plsc.subcore_barrier()

    def _gather_a(j, d):
        return pltpu.make_async_copy(
            y.at[rows2d.at[j, pl.ds(0, MCH // 2)]],
            gbuf.at[d, pl.ds(0, MCH // 2)], gsem.at[d])

    def _gather_b(j, d):
        return pltpu.make_async_copy(
            y.at[rows2d.at[j, pl.ds(MCH // 2, MCH // 2)]],
            gbuf.at[d, pl.ds(MCH // 2, MCH // 2)], gsem2.at[d])

    def _scatter(j, d):
        return pltpu.make_async_copy(
            gbuf.at[d], acc.at[cols2d.at[j]], ssem.at[d])

    def _phase(p, carry):
        pb = start + p * MPC
        q0 = p * MPC
        last15 = jnp.logical_and(s == 15, p == MPHASES - 1)

        @pl.when(jnp.logical_not(last15))
        def _():
            pltpu.sync_copy(ei3.at[0, pl.ds(pb, MPC)], rows2d)
            pltpu.sync_copy(ei3.at[1, pl.ds(pb, MPC)], cols2d)

        @pl.when(last15)
        def _():
            pltpu.sync_copy(ei3.at[0, pl.ds(pb, 32)], rows2d.at[pl.ds(0, 32)])
            pltpu.sync_copy(ei3.at[1, pl.ds(pb, 32)], cols2d.at[pl.ds(0, 32)])
            pltpu.sync_copy(ei3.at[0, pl.ds(2496, 4)],
                            rows2d.at[pl.ds(32, 4)])
            pltpu.sync_copy(ei3.at[1, pl.ds(2496, 4)],
                            cols2d.at[pl.ds(32, 4)])

        def _adj(r, cry):
            for v in range(MCH // LANES):
                sl = pl.ds(v * LANES, LANES)
                rows2d[r, sl] = rows2d[r, sl] + off
            return cry

        lax.fori_loop(0, MPC, _adj, 0)

        for j in range(3):
            @pl.when(q0 + j < nch)
            def _(j=j):
                _gather_a(j, j % 4).start()
                _gather_b(j, j % 4).start()

        for j in range(MPC):
            @pl.when(q0 + j < nch)
            def _(j=j):
                d = j % 4
                _gather_a(j, d).wait()
                _gather_b(j, d).wait()
                _scatter(j, d).start(add=True)
                if j + 3 < MPC:
                    @pl.when(q0 + j + 3 < nch)
                    def _():
                        if j >= 1:
                            _scatter(0, (j - 1) % 4).wait()
                        _gather_a(j + 3, (j + 3) % 4).start()
                        _gather_b(j + 3, (j + 3) % 4).start()

        for dd in range(4):
            _scatter(0, dd).wait()
        return carry

    lax.fori_loop(0, MPHASES, _phase, 0)
    plsc.subcore_barrier()

    pltpu.sync_copy(dis.at[pl.ds(n0, NPS)], disv)
    pltpu.sync_copy(b.at[pl.ds(c * HALF, HALF)], bb)

    def _aread(k, d):
        return pltpu.make_async_copy(
            acc.at[pl.ds(n0 + k * MCH, MCH)], gbuf.at[d], gsem.at[d])

    def _owrite_full(k, d):
        return pltpu.make_async_copy(
            gbuf.at[2 + d],
            outp.at[pl.ds(n0 + k * MCH, MCH), pl.ds(c * HALF, HALF)],
            ssem.at[d])

    def _owrite_part(k, d):
        return pltpu.make_async_copy(
            gbuf.at[2 + d, pl.ds(0, N % MCH)],
            outp.at[pl.ds(n0 + k * MCH, N % MCH), pl.ds(c * HALF, HALF)],
            ssem.at[d])

    def _ostart(k, d):
        ws = n0 + k * MCH

        @pl.when(ws + MCH <= N)
        def _():
            _owrite_full(k, d).start()

        @pl.when(jnp.logical_and(ws < N, ws + MCH > N))
        def _():
            _owrite_part(k, d).start()

    def _owait(k, d):
        ws = n0 + k * MCH

        @pl.when(ws + MCH <= N)
        def _():
            _owrite_full(k, d).wait()

        @pl.when(jnp.logical_and(ws < N, ws + MCH > N))
        def _():
            _owrite_part(k, d).wait()

    _aread(0, 0).start()

    def _finpair(k2, carry):
        for d in range(2):
            k = k2 * 2 + d
            _aread(k, d).wait()

            @pl.when(k + 1 < NB)
            def _(d=d, k=k):
                _aread(k + 1, 1 - d).start()

            @pl.when(k >= 2)
            def _(d=d, k=k):
                _owait(k - 2, d)

            def _fin(g, cry, d=d, k=k):
                dvec = disv[pl.ds(k * MCH + g * LANES, LANES)]
                for i in range(LANES):
                    nn = g * LANES + i
                    dval = dvec[i]
                    for v in range(HALF // LANES):
                        sl = pl.ds(v * LANES, LANES)
                        gbuf[2 + d, nn, sl] = jnp.maximum(
                            gbuf[d, nn, sl] * dval + bb[sl], 0.0)
                return cry

            lax.fori_loop(0, MCH // LANES, _fin, 0)
            _ostart(k, d)
        return carry

    lax.fori_loop(0, NB // 2, _finpair, 0)
    _owait(NB - 2, 0)
    _owait(NB - 1, 1)


def kernel(x, edge_index, W, b):
    ei3 = edge_index.reshape(2, MSG_CHUNKS, MCH)
    dega = _deg(ei3)
    y3, dis = _mm(x, W, dega.T)
    y = y3.reshape(NC * N_PAD, HALF)
    return _msg(ei3, y, dis.reshape(N_PAD), b)
